# baseline (speedup 1.0000x reference)
import jax
import jax.numpy as jnp
from jax.experimental import pallas as pl
from jax.experimental.pallas import tpu as pltpu

_ROWS = 16384
_COLS = 1000
_BLOCK_ROWS = 512
_GRID = _ROWS // _BLOCK_ROWS


def _mse_body(yt_ref, yp_ref, out_ref, acc_ref):
    i = pl.program_id(0)

    @pl.when(i == 0)
    def _init():
        acc_ref[0] = 0.0
        acc_ref[1] = 0.0

    yt = yt_ref[...]
    yp = yp_ref[...]
    mask = (yt != 0.0) & (yp != 0.0)
    d = yt - yp
    sq = jnp.where(mask, d * d, 0.0)
    acc_ref[0] += jnp.sum(sq)
    acc_ref[1] += jnp.sum(mask.astype(jnp.float32))

    @pl.when(i == _GRID - 1)
    def _fin():
        out_ref[0, 0] = acc_ref[0] / acc_ref[1]


def kernel(y_true, y_pred):
    out = pl.pallas_call(
        _mse_body,
        grid=(_GRID,),
        in_specs=[
            pl.BlockSpec((_BLOCK_ROWS, _COLS), lambda i: (i, 0)),
            pl.BlockSpec((_BLOCK_ROWS, _COLS), lambda i: (i, 0)),
        ],
        out_specs=pl.BlockSpec(
            (1, 1), lambda i: (0, 0), memory_space=pltpu.SMEM
        ),
        out_shape=jax.ShapeDtypeStruct((1, 1), jnp.float32),
        scratch_shapes=[pltpu.SMEM((2,), jnp.float32)],
    )(y_true, y_pred)
    return out[0, 0]


# speedup vs baseline: 1.0614x; 1.0614x over previous
import jax
import jax.numpy as jnp
from jax.experimental import pallas as pl
from jax.experimental.pallas import tpu as pltpu

_ROWS = 16384
_COLS = 1000
_BLOCK_ROWS = 2048
_GRID = _ROWS // _BLOCK_ROWS


def _mse_body(yt_ref, yp_ref, out_ref, acc_ref):
    i = pl.program_id(0)

    @pl.when(i == 0)
    def _init():
        acc_ref[0] = 0.0
        acc_ref[1] = 0.0

    yt = yt_ref[...]
    yp = yp_ref[...]
    mask = (yt != 0.0) & (yp != 0.0)
    d = yt - yp
    sq = jnp.where(mask, d * d, 0.0)
    acc_ref[0] += jnp.sum(sq)
    acc_ref[1] += jnp.sum(mask.astype(jnp.float32))

    @pl.when(i == _GRID - 1)
    def _fin():
        out_ref[0, 0] = acc_ref[0] / acc_ref[1]


def kernel(y_true, y_pred):
    out = pl.pallas_call(
        _mse_body,
        grid=(_GRID,),
        in_specs=[
            pl.BlockSpec((_BLOCK_ROWS, _COLS), lambda i: (i, 0)),
            pl.BlockSpec((_BLOCK_ROWS, _COLS), lambda i: (i, 0)),
        ],
        out_specs=pl.BlockSpec(
            (1, 1), lambda i: (0, 0), memory_space=pltpu.SMEM
        ),
        out_shape=jax.ShapeDtypeStruct((1, 1), jnp.float32),
        scratch_shapes=[pltpu.SMEM((2,), jnp.float32)],
    )(y_true, y_pred)
    return out[0, 0]


# speedup vs baseline: 1.0633x; 1.0018x over previous
import jax
import jax.numpy as jnp
from jax.experimental import pallas as pl
from jax.experimental.pallas import tpu as pltpu

_ROWS = 16384
_COLS = 1000
_NSTREAM = 4
_BLOCK_ROWS = 512
_GRID = _ROWS // (_BLOCK_ROWS * _NSTREAM)


def _mse_body(*refs):
    in_refs = refs[: 2 * _NSTREAM]
    out_ref = refs[2 * _NSTREAM]
    acc_ref = refs[2 * _NSTREAM + 1]
    i = pl.program_id(0)

    @pl.when(i == 0)
    def _init():
        acc_ref[0] = 0.0
        acc_ref[1] = 0.0

    tot = 0.0
    cnt = 0.0
    for k in range(_NSTREAM):
        yt = in_refs[k][...]
        yp = in_refs[_NSTREAM + k][...]
        mask = (yt != 0.0) & (yp != 0.0)
        d = yt - yp
        sq = jnp.where(mask, d * d, 0.0)
        tot += jnp.sum(sq)
        cnt += jnp.sum(mask.astype(jnp.float32))
    acc_ref[0] += tot
    acc_ref[1] += cnt

    @pl.when(i == _GRID - 1)
    def _fin():
        out_ref[0, 0] = acc_ref[0] / acc_ref[1]


def kernel(y_true, y_pred):
    specs = [
        pl.BlockSpec((_BLOCK_ROWS, _COLS), lambda i, k=k: (i * _NSTREAM + k, 0))
        for k in range(_NSTREAM)
    ]
    out = pl.pallas_call(
        _mse_body,
        grid=(_GRID,),
        in_specs=specs + specs,
        out_specs=pl.BlockSpec(
            (1, 1), lambda i: (0, 0), memory_space=pltpu.SMEM
        ),
        out_shape=jax.ShapeDtypeStruct((1, 1), jnp.float32),
        scratch_shapes=[pltpu.SMEM((2,), jnp.float32)],
    )(*([y_true] * _NSTREAM + [y_pred] * _NSTREAM))
    return out[0, 0]


# speedup vs baseline: 1.0816x; 1.0172x over previous
import jax
import jax.numpy as jnp
from jax.experimental import pallas as pl
from jax.experimental.pallas import tpu as pltpu

_ROWS = 16384
_COLS = 1000
_CH = 512
_NCH = _ROWS // _CH
_NBUF = 8
_DEPTH = 7


def _mse_body(yt_hbm, yp_hbm, out_ref, bt, bp, semt, semp):
    def copies(j):
        s = j % _NBUF
        rows = pl.ds(j * _CH, _CH)
        return (
            pltpu.make_async_copy(yt_hbm.at[rows, :], bt.at[s], semt.at[s]),
            pltpu.make_async_copy(yp_hbm.at[rows, :], bp.at[s], semp.at[s]),
        )

    for j in range(_DEPTH):
        for c in copies(j):
            c.start()

    tot = jnp.float32(0.0)
    cnt = jnp.float32(0.0)
    for j in range(_NCH):
        for c in copies(j):
            c.wait()
        if j + _DEPTH < _NCH:
            for c in copies(j + _DEPTH):
                c.start()
        yt = bt[j % _NBUF]
        yp = bp[j % _NBUF]
        mask = (yt != 0.0) & (yp != 0.0)
        d = yt - yp
        tot += jnp.sum(jnp.where(mask, d * d, 0.0))
        cnt += jnp.sum(mask.astype(jnp.float32))
    out_ref[0, 0] = tot / cnt


def kernel(y_true, y_pred):
    out = pl.pallas_call(
        _mse_body,
        in_specs=[
            pl.BlockSpec(memory_space=pl.ANY),
            pl.BlockSpec(memory_space=pl.ANY),
        ],
        out_specs=pl.BlockSpec(memory_space=pltpu.SMEM),
        out_shape=jax.ShapeDtypeStruct((1, 1), jnp.float32),
        scratch_shapes=[
            pltpu.VMEM((_NBUF, _CH, _COLS), jnp.float32),
            pltpu.VMEM((_NBUF, _CH, _COLS), jnp.float32),
            pltpu.SemaphoreType.DMA((_NBUF,)),
            pltpu.SemaphoreType.DMA((_NBUF,)),
        ],
    )(y_true, y_pred)
    return out[0, 0]
